# baseline (speedup 1.0000x reference)
import functools

import jax
import jax.numpy as jnp
from jax import lax
from jax.experimental import pallas as pl
from jax.experimental.pallas import tpu as pltpu
from jax.experimental.pallas import tpu_sc as plsc

NUM_MOVES = 1000
EMBED_DIM = 64
BATCH = 16384
NUM_POKEMON = 6
NUM_MOVE_SLOTS = 4

PS = NUM_POKEMON * NUM_MOVE_SLOTS
DT = EMBED_DIM // 8
BT = BATCH // 128
TAB_WORDS = NUM_MOVES * EMBED_DIM

NUM_CORES = 2
NUM_SUBCORES = 16
NUM_WORKERS = NUM_CORES * NUM_SUBCORES
BT_PER_WORKER = BT // NUM_WORKERS
BLOCKS = PS * BT_PER_WORKER


@functools.partial(
    pl.kernel,
    out_type=jax.ShapeDtypeStruct((PS * DT, BT, 1024), jnp.float32),
    mesh=plsc.VectorSubcoreMesh(core_axis_name="c", subcore_axis_name="s"),
    compiler_params=pltpu.CompilerParams(use_tc_tiling_on_sc=False,
                                         needs_layout_passes=False),
    scratch_types=[
        pltpu.VMEM((TAB_WORDS,), jnp.float32),
        pltpu.VMEM((NUM_POKEMON, BT_PER_WORKER, NUM_MOVE_SLOTS, 128),
                   jnp.int32),
        pltpu.VMEM((DT, 1024), jnp.float32),
        pltpu.VMEM((DT, 1024), jnp.float32),
        pltpu.SemaphoreType.DMA,
        pltpu.SemaphoreType.DMA,
    ],
)
def _embed_kernel(idx_hbm, tab_hbm, out_hbm, tab_v, idx_v, buf0, buf1,
                  sem0, sem1):
    wid = lax.axis_index("s") * NUM_CORES + lax.axis_index("c")
    bt0 = wid * BT_PER_WORKER

    pltpu.sync_copy(tab_hbm, tab_v)
    pltpu.sync_copy(idx_hbm.at[:, pl.ds(bt0, BT_PER_WORKER)], idx_v)

    bufs = (buf0, buf1)
    sems = (sem0, sem1)

    def compute_block(blk, buf):
        ps = blk // BT_PER_WORKER
        bt = blk % BT_PER_WORKER
        p = ps // NUM_MOVE_SLOTS
        s = ps % NUM_MOVE_SLOTS
        for g in range(8):
            v = idx_v[p, bt, s, pl.ds(g * 16, 16)]
            vbase = v * EMBED_DIM
            for dt in range(DT):
                for dl in range(8):
                    val = plsc.load_gather(tab_v, [vbase + (dt * 8 + dl)])
                    buf[dt, pl.ds(dl * 128 + g * 16, 16)] = val

    def fire_writeback(blk, buf, sem):
        ps = blk // BT_PER_WORKER
        bt = blk % BT_PER_WORKER
        pltpu.async_copy(
            buf, out_hbm.at[pl.ds(ps * DT, DT), bt0 + bt], sem)

    def wait_writeback(buf, sem):
        pltpu.make_async_copy(
            buf, out_hbm.at[pl.ds(0, DT), 0], sem).wait()

    def body(t, _):
        for u in range(2):
            blk = t * 2 + u

            @pl.when(t > 0)
            def _():
                wait_writeback(bufs[u], sems[u])

            compute_block(blk, bufs[u])
            fire_writeback(blk, bufs[u], sems[u])
        return ()

    lax.fori_loop(0, BLOCKS // 2, body, ())

    for u in range(2):
        wait_writeback(bufs[u], sems[u])


def kernel(move_ids, table):
    i1 = move_ids.astype(jnp.int32).reshape(128, 128, NUM_POKEMON,
                                            NUM_MOVE_SLOTS)
    idx_lin = jnp.transpose(i1, (2, 0, 3, 1)).reshape(
        NUM_POKEMON, 128, NUM_MOVE_SLOTS, 128)
    out_lin = _embed_kernel(idx_lin, table.reshape(TAB_WORDS))
    o1 = out_lin.reshape(NUM_POKEMON, NUM_MOVE_SLOTS, DT, 128, 8, 128)
    o2 = jnp.transpose(o1, (3, 5, 0, 1, 2, 4))
    return o2.reshape(BATCH, NUM_POKEMON, NUM_MOVE_SLOTS, EMBED_DIM)


# speedup vs baseline: 7.9059x; 7.9059x over previous
import functools

import jax
import jax.numpy as jnp
from jax import lax
from jax.experimental import pallas as pl
from jax.experimental.pallas import tpu as pltpu
from jax.experimental.pallas import tpu_sc as plsc

NUM_MOVES = 1000
EMBED_DIM = 64
BATCH = 16384
NUM_POKEMON = 6
NUM_MOVE_SLOTS = 4

PS = NUM_POKEMON * NUM_MOVE_SLOTS
DT = EMBED_DIM // 8
BT = BATCH // 128
TAB_WORDS = NUM_MOVES * EMBED_DIM

NUM_CORES = 2
NUM_SUBCORES = 16
NUM_WORKERS = NUM_CORES * NUM_SUBCORES
BT_PER_WORKER = BT // NUM_WORKERS
BLOCKS = PS * BT_PER_WORKER


@functools.partial(
    pl.kernel,
    out_type=jax.ShapeDtypeStruct((PS * DT, BT, 1024), jnp.float32),
    mesh=plsc.VectorSubcoreMesh(core_axis_name="c", subcore_axis_name="s"),
    compiler_params=pltpu.CompilerParams(use_tc_tiling_on_sc=False,
                                         needs_layout_passes=False),
    scratch_types=[
        pltpu.VMEM((TAB_WORDS,), jnp.float32),
        pltpu.VMEM((NUM_POKEMON, BT_PER_WORKER, NUM_MOVE_SLOTS, 128),
                   jnp.int32),
        pltpu.VMEM((DT, 1024), jnp.float32),
        pltpu.VMEM((DT, 1024), jnp.float32),
        pltpu.SemaphoreType.DMA,
        pltpu.SemaphoreType.DMA,
    ],
)
def _embed_kernel(idx_hbm, tab_hbm, out_hbm, tab_v, idx_v, buf0, buf1,
                  sem0, sem1):
    wid = lax.axis_index("s") * NUM_CORES + lax.axis_index("c")
    bt0 = wid * BT_PER_WORKER

    pltpu.sync_copy(tab_hbm, tab_v)
    pltpu.sync_copy(idx_hbm.at[:, pl.ds(bt0, BT_PER_WORKER)], idx_v)

    bufs = (buf0, buf1)
    sems = (sem0, sem1)

    tab_slc = [tab_v.at[pl.ds(k * NUM_MOVES, NUM_MOVES)]
               for k in range(EMBED_DIM)]
    PIPE = 6

    def compute_block(blk, buf):
        ps = blk // BT_PER_WORKER
        bt = blk % BT_PER_WORKER
        p = ps // NUM_MOVE_SLOTS
        s = ps % NUM_MOVE_SLOTS
        for g in range(8):
            v = idx_v[p, bt, s, pl.ds(g * 16, 16)]
            vals = [None] * EMBED_DIM
            for k in range(EMBED_DIM + PIPE):
                if k < EMBED_DIM:
                    vals[k] = plsc.load_gather(tab_slc[k], [v])
                if k >= PIPE:
                    kk = k - PIPE
                    dt, dl = kk // 8, kk % 8
                    buf[dt, pl.ds(dl * 128 + g * 16, 16)] = vals[kk]

    def fire_writeback(blk, buf, sem):
        ps = blk // BT_PER_WORKER
        bt = blk % BT_PER_WORKER
        pltpu.async_copy(
            buf, out_hbm.at[pl.ds(ps * DT, DT), bt0 + bt], sem)

    def wait_writeback(buf, sem):
        pltpu.make_async_copy(
            buf, out_hbm.at[pl.ds(0, DT), 0], sem).wait()

    def body(t, _):
        for u in range(2):
            blk = t * 2 + u

            @pl.when(t > 0)
            def _():
                wait_writeback(bufs[u], sems[u])

            compute_block(blk, bufs[u])
            fire_writeback(blk, bufs[u], sems[u])
        return ()

    lax.fori_loop(0, BLOCKS // 2, body, ())

    for u in range(2):
        wait_writeback(bufs[u], sems[u])


def kernel(move_ids, table):
    i1 = move_ids.astype(jnp.int32).reshape(128, 128, NUM_POKEMON,
                                            NUM_MOVE_SLOTS)
    idx_lin = jnp.transpose(i1, (2, 0, 3, 1)).reshape(
        NUM_POKEMON, 128, NUM_MOVE_SLOTS, 128)
    out_lin = _embed_kernel(idx_lin, table.T.reshape(TAB_WORDS))
    o1 = out_lin.reshape(NUM_POKEMON, NUM_MOVE_SLOTS, DT, 128, 8, 128)
    o2 = jnp.transpose(o1, (3, 5, 0, 1, 2, 4))
    return o2.reshape(BATCH, NUM_POKEMON, NUM_MOVE_SLOTS, EMBED_DIM)
